# baseline (speedup 1.0000x reference)
import functools

import jax
import jax.numpy as jnp
from jax import lax
from jax.experimental import pallas as pl
from jax.experimental.pallas import tpu as pltpu
from jax.experimental.pallas import tpu_sc as plsc

D = 256
NC = 2
NS = 16
NW = NC * NS
PER_W = 6272
WIN = 64
KWIN = 2000
MAXW = PER_W // WIN


def _sc_kernel(keys_hbm, pe_hbm, out_hbm,
               winner_v, kbuf, ibuf, inv_flat, zbuf, sem):
    wid = lax.axis_index("s") * NC + lax.axis_index("c")
    rows_total = out_hbm.shape[0]
    nkeys = keys_hbm.shape[0]
    base = wid * PER_W
    nrows = jnp.minimum(PER_W, rows_total - base)
    hi = base + nrows
    lanes = lax.iota(jnp.int32, 16)

    scope_a = jax.named_scope("phase_a_winner")
    scope_a.__enter__()

    def init_body(t, _):
        winner_v[pl.ds(t * 16, 16)] = jnp.full((16,), -1, jnp.int32)
        return 0
    lax.fori_loop(0, PER_W // 16, init_body, 0)

    def key_window(w, _):
        pltpu.sync_copy(keys_hbm.at[pl.ds(w * KWIN, KWIN)], kbuf)

        def chunk(t, _):
            kv = kbuf[pl.ds(t * 16, 16)]
            iv = (w * KWIN + t * 16) + lanes
            _, last = plsc.scan_count(kv)
            m = last & (kv >= base) & (kv < hi)
            plsc.store_scatter(winner_v, [kv - base], iv, mask=m)
            return 0
        lax.fori_loop(0, KWIN // 16, chunk, 0)
        return 0
    lax.fori_loop(0, nkeys // KWIN, key_window, 0)
    scope_a.__exit__(None, None, None)
    scope_b = jax.named_scope("phase_b_remap")
    scope_b.__enter__()

    def remap(t, off):
        wv = winner_v[pl.ds(t * 16, 16)]
        rowv = (base + t * 16) + lanes
        inv = wv < 0
        in_slab = rowv < hi
        ibuf[pl.ds(t * 16, 16)] = jnp.where(inv, rowv, wv)
        mm = inv & in_slab
        plsc.store_compressed(inv_flat.at[pl.ds(off, 16)], rowv, mask=mm)
        cnt = jnp.max(plsc.all_reduce_population_count(mm))
        return off + cnt
    cnt = lax.fori_loop(0, PER_W // 16, remap, jnp.int32(0))
    scope_b.__exit__(None, None, None)
    scope_g = jax.named_scope("phase_c_gather")
    scope_g.__enter__()

    def gwin(w, _):
        st = base + w * WIN
        pltpu.async_copy(pe_hbm.at[ibuf.at[pl.ds(w * WIN, WIN)]],
                         zbuf, sem).wait()
        pltpu.sync_copy(zbuf, out_hbm.at[pl.ds(st, WIN), :])
        return 0
    lax.fori_loop(0, nrows // WIN, gwin, 0)
    scope_g.__exit__(None, None, None)
    scope_f = jax.named_scope("phase_d_fixup")
    scope_f.__enter__()

    def zrow(t, _):
        zbuf[t // 16, pl.ds((t % 16) * 16, 16)] = jnp.zeros((16,), jnp.float32)
        return 0
    lax.fori_loop(0, WIN * (D // 16), zrow, 0)

    zsrc = zbuf.at[pl.ds(0, 16), :]
    nfull = cnt // 16
    r = cnt % 16

    def zwin(w, _):
        v = inv_flat[pl.ds(w * 16, 16)]
        pltpu.async_copy(zsrc, out_hbm.at[v], sem).wait()
        return 0
    lax.fori_loop(0, nfull, zwin, 0)

    @pl.when(r > 0)
    def _tail():
        v = inv_flat[pl.ds(nfull * 16, 16)]
        bvec = lax.gather(
            v, jnp.full((16, 1), r - 1, jnp.int32),
            dimension_numbers=lax.GatherDimensionNumbers(
                offset_dims=(), collapsed_slice_dims=(0,),
                start_index_map=(0,)),
            slice_sizes=(1,),
            mode=lax.GatherScatterMode.PROMISE_IN_BOUNDS)
        vfinal = jnp.where(lanes < r, v, bvec)
        pltpu.async_copy(zsrc, out_hbm.at[vfinal], sem).wait()

    scope_f.__exit__(None, None, None)


@functools.partial(jax.jit, static_argnums=())
def _sc_call(keys, pe):
    rows = keys.shape[0]
    call = functools.partial(
        pl.kernel,
        out_type=jax.ShapeDtypeStruct((rows, D), jnp.float32),
        mesh=plsc.VectorSubcoreMesh(core_axis_name="c", subcore_axis_name="s"),
        compiler_params=pltpu.CompilerParams(needs_layout_passes=False),
        scratch_types=[
            pltpu.VMEM((PER_W,), jnp.int32),
            pltpu.VMEM((KWIN,), jnp.int32),
            pltpu.VMEM((PER_W,), jnp.int32),
            pltpu.VMEM((PER_W + WIN,), jnp.int32),
            pltpu.VMEM((WIN, D), jnp.float32),
            pltpu.SemaphoreType.DMA,
        ],
    )(_sc_kernel)
    return call(keys, pe)


def kernel(x, pe):
    rows = x.shape[0]
    n = rows // 2
    keys = x[:, 0] * 2 + x[:, 1]
    out = _sc_call(keys, pe)
    return out.reshape(n, 2 * D)


# speedup vs baseline: 1.0870x; 1.0870x over previous
import functools

import jax
import jax.numpy as jnp
from jax import lax
from jax.experimental import pallas as pl
from jax.experimental.pallas import tpu as pltpu
from jax.experimental.pallas import tpu_sc as plsc

D = 256
NC = 2
NS = 16
NW = NC * NS
PER_W = 6272
WIN = 32
KWIN = 4000


def _sc_kernel(keys_hbm, pe_hbm, out_hbm,
               winner_v, kbufs, ibuf, vbuf, bufs, ksems, gsems, ssems):
    wid = lax.axis_index("s") * NC + lax.axis_index("c")
    nslots = 2 * out_hbm.shape[0]
    nkeys = keys_hbm.shape[0]
    base = wid * PER_W
    nrows = jnp.minimum(PER_W, nslots - base)
    hi = base + nrows
    lanes = lax.iota(jnp.int32, 16)
    zeros16 = jnp.zeros((16,), jnp.float32)

    scope_a = jax.named_scope("phase_a_winner")
    scope_a.__enter__()

    def init_body(t, _):
        winner_v[pl.ds(t * 16, 16)] = jnp.full((16,), -1, jnp.int32)
        return 0
    lax.fori_loop(0, PER_W // 16, init_body, 0)

    nk = nkeys // KWIN

    def kload(w, p):
        pltpu.async_copy(keys_hbm.at[pl.ds(w * KWIN, KWIN)], kbufs[p],
                         ksems[p])

    def kwait(p):
        pltpu.make_async_copy(keys_hbm.at[pl.ds(0, KWIN)], kbufs[p],
                              ksems[p]).wait()

    def kproc(w, p):
        kbuf = kbufs[p]

        def chunk(t, _):
            kv = kbuf[pl.ds(t * 16, 16)]
            mr = (kv >= base) & (kv < hi)
            nin = jnp.max(plsc.all_reduce_population_count(mr))

            @pl.when(nin == 1)
            def _single():
                iv = (w * KWIN + t * 16) + lanes
                plsc.store_scatter(winner_v, [kv - base], iv, mask=mr)

            @pl.when(nin > 1)
            def _multi():
                iv = (w * KWIN + t * 16) + lanes
                _, last = plsc.scan_count(kv)
                plsc.store_scatter(winner_v, [kv - base], iv,
                                   mask=mr & last)
            return 0
        lax.fori_loop(0, KWIN // 16, chunk, 0)

    kload(0, 0)

    def kpipe(i2, _):
        w0 = i2 * 2
        kwait(0)

        @pl.when(w0 + 1 < nk)
        def _():
            kload(w0 + 1, 1)
        kproc(w0, 0)
        kwait(1)

        @pl.when(w0 + 2 < nk)
        def _():
            kload(w0 + 2, 0)
        kproc(w0 + 1, 1)
        return 0
    lax.fori_loop(0, nk // 2, kpipe, 0)
    scope_a.__exit__(None, None, None)

    def _dg(v, idxvec):
        return lax.gather(
            v, idxvec[:, None],
            dimension_numbers=lax.GatherDimensionNumbers(
                offset_dims=(), collapsed_slice_dims=(0,),
                start_index_map=(0,)),
            slice_sizes=(1,),
            mode=lax.GatherScatterMode.PROMISE_IN_BOUNDS)

    emap = (lanes % 8) * 2
    omap = emap + 1
    half = lanes < 8

    def remap(w, _):
        wa = winner_v[pl.ds(w * WIN, 16)]
        wb = winner_v[pl.ds(w * WIN + 16, 16)]
        ra = (base + w * WIN) + lanes
        rb = ra + 16
        ia = jnp.where(wa < 0, ra, wa)
        ib = jnp.where(wb < 0, rb, wb)
        evens = jnp.where(half, _dg(ia, emap), _dg(ib, emap))
        odds = jnp.where(half, _dg(ia, omap), _dg(ib, omap))
        ibuf[pl.ds(w * WIN, 16)] = evens
        ibuf[pl.ds(w * WIN + 16, 16)] = odds
        vbuf[pl.ds(w * WIN, 16)] = jnp.where(half, _dg(wa, emap),
                                             _dg(wb, emap))
        vbuf[pl.ds(w * WIN + 16, 16)] = jnp.where(half, _dg(wa, omap),
                                                  _dg(wb, omap))
        return 0
    lax.fori_loop(0, PER_W // WIN, remap, 0)

    scope_g = jax.named_scope("phase_b_gather")
    scope_g.__enter__()
    nwin = nrows // WIN

    def gissue(w, p):
        pltpu.async_copy(pe_hbm.at[ibuf.at[pl.ds(w * WIN, WIN)]], bufs[p],
                         gsems[p])

    def gwait(p):
        pltpu.make_async_copy(pe_hbm.at[ibuf.at[pl.ds(0, WIN)]], bufs[p],
                              gsems[p]).wait()

    def sissue(w, p):
        node_st = pl.multiple_of((base + w * WIN) // 2, 8)
        pltpu.async_copy(bufs[p].at[pl.ds(0, WIN // 2), :],
                         out_hbm.at[pl.ds(node_st, WIN // 2), pl.ds(0, D)],
                         ssems[p])
        pltpu.async_copy(bufs[p].at[pl.ds(WIN // 2, WIN // 2), :],
                         out_hbm.at[pl.ds(node_st, WIN // 2), pl.ds(D, D)],
                         ssems[p])

    def swait(p):
        for _ in range(2):
            pltpu.make_async_copy(
                bufs[p].at[pl.ds(0, WIN // 2), :],
                out_hbm.at[pl.ds(0, WIN // 2), pl.ds(0, D)],
                ssems[p]).wait()

    def zerofix(w, p):
        buf = bufs[p]
        for c in range(WIN // 16):
            wv = vbuf[pl.ds(w * WIN + c * 16, 16)]
            bits = jnp.sum(jnp.where(wv < 0, jnp.int32(1), jnp.int32(0))
                           << lanes)

            @pl.when(bits != 0)
            def _():
                def zrow(l, _):
                    @pl.when(((bits >> l) & 1) == 1)
                    def _():
                        for k in range(D // 16):
                            buf[c * 16 + l, pl.ds(k * 16, 16)] = zeros16
                    return 0
                lax.fori_loop(0, 16, zrow, 0)

    gissue(0, 0)

    def pipe(i2, _):
        w0 = i2 * 2
        gwait(0)

        @pl.when(w0 >= 1)
        def _():
            swait(1)
        gissue(w0 + 1, 1)
        zerofix(w0, 0)
        sissue(w0, 0)

        gwait(1)
        swait(0)

        @pl.when(w0 + 2 < nwin)
        def _():
            gissue(w0 + 2, 0)
        zerofix(w0 + 1, 1)
        sissue(w0 + 1, 1)
        return 0
    lax.fori_loop(0, nwin // 2, pipe, 0)
    swait(1)
    scope_g.__exit__(None, None, None)


def _sc_call(keys, pe):
    nodes = keys.shape[0] // 2
    call = functools.partial(
        pl.kernel,
        out_type=jax.ShapeDtypeStruct((nodes, 2 * D), jnp.float32),
        mesh=plsc.VectorSubcoreMesh(core_axis_name="c", subcore_axis_name="s"),
        compiler_params=pltpu.CompilerParams(needs_layout_passes=False),
        scratch_types=[
            pltpu.VMEM((PER_W,), jnp.int32),
            [pltpu.VMEM((KWIN,), jnp.int32)] * 2,
            pltpu.VMEM((PER_W,), jnp.int32),
            pltpu.VMEM((PER_W,), jnp.int32),
            [pltpu.VMEM((WIN, D), jnp.float32)] * 2,
            [pltpu.SemaphoreType.DMA] * 2,
            [pltpu.SemaphoreType.DMA] * 2,
            [pltpu.SemaphoreType.DMA] * 2,
        ],
    )(_sc_kernel)
    return call(keys, pe)


def kernel(x, pe):
    keys = x[:, 0] * 2 + x[:, 1]
    return _sc_call(keys, pe)


# speedup vs baseline: 2.8460x; 2.6183x over previous
import functools

import jax
import jax.numpy as jnp
from jax import lax
from jax.experimental import pallas as pl
from jax.experimental.pallas import tpu as pltpu
from jax.experimental.pallas import tpu_sc as plsc

D = 256
NC = 2
NS = 16
NW = NC * NS
PER_W = 6272
WIN = 32
KWIN = 4000
NBUF = 4


def _sc_kernel(keys_hbm, pe_hbm, out_hbm,
               winner_v, kbufs, ibuf, vbuf, bufs, ksems, gsems, ssems):
    wid = lax.axis_index("s") * NC + lax.axis_index("c")
    nslots = 2 * out_hbm.shape[0]
    nkeys = keys_hbm.shape[0]
    base = wid * PER_W
    nrows = jnp.minimum(PER_W, nslots - base)
    hi = base + nrows
    lanes = lax.iota(jnp.int32, 16)
    zeros16 = jnp.zeros((16,), jnp.float32)

    scope_a = jax.named_scope("phase_a_winner")
    scope_a.__enter__()

    def init_body(t, _):
        winner_v[pl.ds(t * 16, 16)] = jnp.full((16,), -1, jnp.int32)
        return 0
    lax.fori_loop(0, PER_W // 16, init_body, 0)

    nk = nkeys // KWIN

    def kload(w, p):
        pltpu.async_copy(keys_hbm.at[pl.ds(w * KWIN, KWIN)], kbufs[p],
                         ksems[p])

    def kwait(p):
        pltpu.make_async_copy(keys_hbm.at[pl.ds(0, KWIN)], kbufs[p],
                              ksems[p]).wait()

    def kproc(w, p):
        kbuf = kbufs[p]

        def chunk5(t5, _):
            for u in range(5):
                t = t5 * 5 + u
                kv = kbuf[pl.ds(t * 16, 16)]
                mr = (kv >= base) & (kv < hi)
                iv = (w * KWIN + t * 16) + lanes
                plsc.store_scatter(winner_v, [kv - base], iv, mask=mr)
            return 0
        lax.fori_loop(0, KWIN // 80, chunk5, 0)

    kload(0, 0)

    def kpipe(i2, _):
        w0 = i2 * 2
        kwait(0)

        @pl.when(w0 + 1 < nk)
        def _():
            kload(w0 + 1, 1)
        kproc(w0, 0)
        kwait(1)

        @pl.when(w0 + 2 < nk)
        def _():
            kload(w0 + 2, 0)
        kproc(w0 + 1, 1)
        return 0
    lax.fori_loop(0, nk // 2, kpipe, 0)
    scope_a.__exit__(None, None, None)

    def _dg(v, idxvec):
        return lax.gather(
            v, idxvec[:, None],
            dimension_numbers=lax.GatherDimensionNumbers(
                offset_dims=(), collapsed_slice_dims=(0,),
                start_index_map=(0,)),
            slice_sizes=(1,),
            mode=lax.GatherScatterMode.PROMISE_IN_BOUNDS)

    emap = (lanes % 8) * 2
    omap = emap + 1
    half = lanes < 8

    def remap(w, _):
        wa = winner_v[pl.ds(w * WIN, 16)]
        wb = winner_v[pl.ds(w * WIN + 16, 16)]
        ra = (base + w * WIN) + lanes
        rb = ra + 16
        ia = jnp.where(wa < 0, ra, wa)
        ib = jnp.where(wb < 0, rb, wb)
        evens = jnp.where(half, _dg(ia, emap), _dg(ib, emap))
        odds = jnp.where(half, _dg(ia, omap), _dg(ib, omap))
        ibuf[pl.ds(w * WIN, 16)] = evens
        ibuf[pl.ds(w * WIN + 16, 16)] = odds
        vbuf[pl.ds(w * WIN, 16)] = jnp.where(half, _dg(wa, emap),
                                             _dg(wb, emap))
        vbuf[pl.ds(w * WIN + 16, 16)] = jnp.where(half, _dg(wa, omap),
                                                  _dg(wb, omap))
        return 0
    lax.fori_loop(0, PER_W // WIN, remap, 0)

    scope_g = jax.named_scope("phase_b_gather")
    scope_g.__enter__()
    nwin = nrows // WIN

    def gissue(w, p):
        pltpu.async_copy(pe_hbm.at[ibuf.at[pl.ds(w * WIN, WIN)]], bufs[p],
                         gsems[p])

    def gwait(p):
        pltpu.make_async_copy(pe_hbm.at[ibuf.at[pl.ds(0, WIN)]], bufs[p],
                              gsems[p]).wait()

    def sissue(w, p):
        node_st = pl.multiple_of((base + w * WIN) // 2, 8)
        pltpu.async_copy(bufs[p].at[pl.ds(0, WIN // 2), :],
                         out_hbm.at[pl.ds(node_st, WIN // 2), pl.ds(0, D)],
                         ssems[p])
        pltpu.async_copy(bufs[p].at[pl.ds(WIN // 2, WIN // 2), :],
                         out_hbm.at[pl.ds(node_st, WIN // 2), pl.ds(D, D)],
                         ssems[p])

    def swait(p):
        for _ in range(2):
            pltpu.make_async_copy(
                bufs[p].at[pl.ds(0, WIN // 2), :],
                out_hbm.at[pl.ds(0, WIN // 2), pl.ds(0, D)],
                ssems[p]).wait()

    def zerofix(w, p):
        buf = bufs[p]
        for c in range(WIN // 16):
            wv = vbuf[pl.ds(w * WIN + c * 16, 16)]
            bits = jnp.sum(jnp.where(wv < 0, jnp.int32(1), jnp.int32(0))
                           << lanes)

            @pl.when(bits != 0)
            def _():
                def zrow(l, _):
                    @pl.when(((bits >> l) & 1) == 1)
                    def _():
                        for k in range(D // 16):
                            buf[c * 16 + l, pl.ds(k * 16, 16)] = zeros16
                    return 0
                lax.fori_loop(0, 16, zrow, 0)

    for p in range(NBUF - 1):
        gissue(p, p)

    def pipe(i, _):
        for p in range(NBUF):
            w = i * NBUF + p

            @pl.when(w < nwin)
            def _(w=w, p=p):
                gwait(p)
                q = (p + NBUF - 1) % NBUF

                @pl.when(w + NBUF - 1 < nwin)
                def _():
                    @pl.when(w >= 1)
                    def _():
                        swait(q)
                    gissue(w + NBUF - 1, q)
                zerofix(w, p)
                sissue(w, p)
        return 0
    lax.fori_loop(0, (PER_W // WIN + NBUF - 1) // NBUF, pipe, 0)
    for p in range(NBUF):
        swait(p)
    scope_g.__exit__(None, None, None)


def _sc_call(keys, pe):
    nodes = keys.shape[0] // 2
    call = functools.partial(
        pl.kernel,
        out_type=jax.ShapeDtypeStruct((nodes, 2 * D), jnp.float32),
        mesh=plsc.VectorSubcoreMesh(core_axis_name="c", subcore_axis_name="s"),
        compiler_params=pltpu.CompilerParams(needs_layout_passes=False),
        scratch_types=[
            pltpu.VMEM((PER_W,), jnp.int32),
            [pltpu.VMEM((KWIN,), jnp.int32)] * 2,
            pltpu.VMEM((PER_W,), jnp.int32),
            pltpu.VMEM((PER_W,), jnp.int32),
            [pltpu.VMEM((WIN, D), jnp.float32)] * NBUF,
            [pltpu.SemaphoreType.DMA] * 2,
            [pltpu.SemaphoreType.DMA] * NBUF,
            [pltpu.SemaphoreType.DMA] * NBUF,
        ],
    )(_sc_kernel)
    return call(keys, pe)


def kernel(x, pe):
    keys = x[:, 0] * 2 + x[:, 1]
    return _sc_call(keys, pe)


# speedup vs baseline: 2.8791x; 1.0116x over previous
import functools

import jax
import jax.numpy as jnp
from jax import lax
from jax.experimental import pallas as pl
from jax.experimental.pallas import tpu as pltpu
from jax.experimental.pallas import tpu_sc as plsc

D = 256
NC = 2
NS = 16
NW = NC * NS
PER_W = 6272
WIN = 64
KWIN = 4000
NBUF = 4


def _sc_kernel(keys_hbm, pe_hbm, out_hbm,
               winner_v, kbufs, ibuf, vbuf, bufs, ksems, gsems, ssems):
    wid = lax.axis_index("s") * NC + lax.axis_index("c")
    nslots = 2 * out_hbm.shape[0]
    nkeys = keys_hbm.shape[0]
    base = wid * PER_W
    nrows = jnp.minimum(PER_W, nslots - base)
    hi = base + nrows
    lanes = lax.iota(jnp.int32, 16)
    zeros16 = jnp.zeros((16,), jnp.float32)

    scope_a = jax.named_scope("phase_a_winner")
    scope_a.__enter__()

    def init_body(t, _):
        winner_v[pl.ds(t * 16, 16)] = jnp.full((16,), -1, jnp.int32)
        return 0
    lax.fori_loop(0, PER_W // 16, init_body, 0)

    nk = nkeys // KWIN

    def kload(w, p):
        pltpu.async_copy(keys_hbm.at[pl.ds(w * KWIN, KWIN)], kbufs[p],
                         ksems[p])

    def kwait(p):
        pltpu.make_async_copy(keys_hbm.at[pl.ds(0, KWIN)], kbufs[p],
                              ksems[p]).wait()

    def kproc(w, p):
        kbuf = kbufs[p]

        nrows_u = nrows.astype(jnp.uint32)

        def chunk5(t5, _):
            for u in range(5):
                t = t5 * 5 + u
                kb = kbuf[pl.ds(t * 16, 16)] - base
                mr = kb.astype(jnp.uint32) < nrows_u
                iv = (w * KWIN + t * 16) + lanes
                plsc.store_scatter(winner_v, [kb], iv, mask=mr)
            return 0
        lax.fori_loop(0, KWIN // 80, chunk5, 0)

    kload(0, 0)

    def kpipe(i2, _):
        w0 = i2 * 2
        kwait(0)

        @pl.when(w0 + 1 < nk)
        def _():
            kload(w0 + 1, 1)
        kproc(w0, 0)
        kwait(1)

        @pl.when(w0 + 2 < nk)
        def _():
            kload(w0 + 2, 0)
        kproc(w0 + 1, 1)
        return 0
    lax.fori_loop(0, nk // 2, kpipe, 0)
    scope_a.__exit__(None, None, None)

    def _dg(v, idxvec):
        return lax.gather(
            v, idxvec[:, None],
            dimension_numbers=lax.GatherDimensionNumbers(
                offset_dims=(), collapsed_slice_dims=(0,),
                start_index_map=(0,)),
            slice_sizes=(1,),
            mode=lax.GatherScatterMode.PROMISE_IN_BOUNDS)

    emap = (lanes % 8) * 2
    omap = emap + 1
    half = lanes < 8

    def remap(w, _):
        for h in range(WIN // 32):
            wa = winner_v[pl.ds(w * WIN + h * 32, 16)]
            wb = winner_v[pl.ds(w * WIN + h * 32 + 16, 16)]
            ra = (base + w * WIN + h * 32) + lanes
            rb = ra + 16
            ia = jnp.where(wa < 0, ra, wa)
            ib = jnp.where(wb < 0, rb, wb)
            evens = jnp.where(half, _dg(ia, emap), _dg(ib, emap))
            odds = jnp.where(half, _dg(ia, omap), _dg(ib, omap))
            ibuf[pl.ds(w * WIN + h * 16, 16)] = evens
            ibuf[pl.ds(w * WIN + WIN // 2 + h * 16, 16)] = odds
            vbuf[pl.ds(w * WIN + h * 16, 16)] = jnp.where(
                half, _dg(wa, emap), _dg(wb, emap))
            vbuf[pl.ds(w * WIN + WIN // 2 + h * 16, 16)] = jnp.where(
                half, _dg(wa, omap), _dg(wb, omap))
        return 0
    lax.fori_loop(0, PER_W // WIN, remap, 0)

    scope_g = jax.named_scope("phase_b_gather")
    scope_g.__enter__()
    nwin = nrows // WIN

    def gissue(w, p):
        pltpu.async_copy(pe_hbm.at[ibuf.at[pl.ds(w * WIN, WIN)]], bufs[p],
                         gsems[p])

    def gwait(p):
        pltpu.make_async_copy(pe_hbm.at[ibuf.at[pl.ds(0, WIN)]], bufs[p],
                              gsems[p]).wait()

    def sissue(w, p):
        node_st = pl.multiple_of((base + w * WIN) // 2, 8)
        pltpu.async_copy(bufs[p].at[pl.ds(0, WIN // 2), :],
                         out_hbm.at[pl.ds(node_st, WIN // 2), pl.ds(0, D)],
                         ssems[p])
        pltpu.async_copy(bufs[p].at[pl.ds(WIN // 2, WIN // 2), :],
                         out_hbm.at[pl.ds(node_st, WIN // 2), pl.ds(D, D)],
                         ssems[p])

    def swait(p):
        for _ in range(2):
            pltpu.make_async_copy(
                bufs[p].at[pl.ds(0, WIN // 2), :],
                out_hbm.at[pl.ds(0, WIN // 2), pl.ds(0, D)],
                ssems[p]).wait()

    def zerofix(w, p):
        buf = bufs[p]
        for c in range(WIN // 16):
            wv = vbuf[pl.ds(w * WIN + c * 16, 16)]
            bits = jnp.sum(jnp.where(wv < 0, jnp.int32(1), jnp.int32(0))
                           << lanes)

            @pl.when(bits != 0)
            def _():
                def zrow(l, _):
                    @pl.when(((bits >> l) & 1) == 1)
                    def _():
                        for k in range(D // 16):
                            buf[c * 16 + l, pl.ds(k * 16, 16)] = zeros16
                    return 0
                lax.fori_loop(0, 16, zrow, 0)

    for p in range(NBUF - 1):
        gissue(p, p)

    def pipe(i, _):
        for p in range(NBUF):
            w = i * NBUF + p

            @pl.when(w < nwin)
            def _(w=w, p=p):
                gwait(p)
                q = (p + NBUF - 1) % NBUF

                @pl.when(w + NBUF - 1 < nwin)
                def _():
                    @pl.when(w >= 1)
                    def _():
                        swait(q)
                    gissue(w + NBUF - 1, q)
                zerofix(w, p)
                sissue(w, p)
        return 0
    lax.fori_loop(0, (PER_W // WIN + NBUF - 1) // NBUF, pipe, 0)
    for p in range(NBUF):
        swait(p)
    scope_g.__exit__(None, None, None)


def _sc_call(keys, pe):
    nodes = keys.shape[0] // 2
    call = functools.partial(
        pl.kernel,
        out_type=jax.ShapeDtypeStruct((nodes, 2 * D), jnp.float32),
        mesh=plsc.VectorSubcoreMesh(core_axis_name="c", subcore_axis_name="s"),
        compiler_params=pltpu.CompilerParams(needs_layout_passes=False),
        scratch_types=[
            pltpu.VMEM((PER_W,), jnp.int32),
            [pltpu.VMEM((KWIN,), jnp.int32)] * 2,
            pltpu.VMEM((PER_W,), jnp.int32),
            pltpu.VMEM((PER_W,), jnp.int32),
            [pltpu.VMEM((WIN, D), jnp.float32)] * NBUF,
            [pltpu.SemaphoreType.DMA] * 2,
            [pltpu.SemaphoreType.DMA] * NBUF,
            [pltpu.SemaphoreType.DMA] * NBUF,
        ],
    )(_sc_kernel)
    return call(keys, pe)


def kernel(x, pe):
    keys = x[:, 0] * 2 + x[:, 1]
    return _sc_call(keys, pe)


# speedup vs baseline: 3.5636x; 1.2377x over previous
import functools

import jax
import jax.numpy as jnp
from jax import lax
from jax.experimental import pallas as pl
from jax.experimental.pallas import tpu as pltpu
from jax.experimental.pallas import tpu_sc as plsc

D = 256
NC = 2
NS = 16
NW = NC * NS
PER_W = 6272
WIN = 64
KWIN = 4000
NBUF = 5


def _sc_kernel(keys_hbm, pe_hbm, out_hbm,
               winner_v, kbufs, ibuf, vbuf, bufs, ksems, gsems, ssems):
    wid = lax.axis_index("s") * NC + lax.axis_index("c")
    nslots = 2 * out_hbm.shape[0]
    nkeys = keys_hbm.shape[0]
    base = wid * PER_W
    nrows = jnp.minimum(PER_W, nslots - base)
    hi = base + nrows
    lanes = lax.iota(jnp.int32, 16)
    zeros16 = jnp.zeros((16,), jnp.float32)

    scope_a = jax.named_scope("phase_a_winner")
    scope_a.__enter__()

    def init_body(t, _):
        winner_v[pl.ds(t * 16, 16)] = jnp.full((16,), -1, jnp.int32)
        return 0
    lax.fori_loop(0, PER_W // 16, init_body, 0)

    nk = nkeys // KWIN

    def kload(w, p):
        pltpu.async_copy(keys_hbm.at[pl.ds(w * KWIN, KWIN)], kbufs[p],
                         ksems[p])

    def kwait(p):
        pltpu.make_async_copy(keys_hbm.at[pl.ds(0, KWIN)], kbufs[p],
                              ksems[p]).wait()

    def kproc(w, p):
        kbuf = kbufs[p]

        nrows_u = nrows.astype(jnp.uint32)
        UNR = 10

        def chunkn(tn, _):
            kbs = [kbuf[pl.ds((tn * UNR + u) * 16, 16)] - base
                   for u in range(UNR)]
            for u in range(UNR):
                t = tn * UNR + u
                mr = kbs[u].astype(jnp.uint32) < nrows_u
                iv = (w * KWIN + t * 16) + lanes
                plsc.store_scatter(winner_v, [kbs[u]], iv, mask=mr)
            return 0
        lax.fori_loop(0, KWIN // (16 * UNR), chunkn, 0)

    kload(0, 0)

    def kpipe(i2, _):
        w0 = i2 * 2
        kwait(0)

        @pl.when(w0 + 1 < nk)
        def _():
            kload(w0 + 1, 1)
        kproc(w0, 0)
        kwait(1)

        @pl.when(w0 + 2 < nk)
        def _():
            kload(w0 + 2, 0)
        kproc(w0 + 1, 1)
        return 0
    lax.fori_loop(0, nk // 2, kpipe, 0)
    scope_a.__exit__(None, None, None)

    def _dg(v, idxvec):
        return lax.gather(
            v, idxvec[:, None],
            dimension_numbers=lax.GatherDimensionNumbers(
                offset_dims=(), collapsed_slice_dims=(0,),
                start_index_map=(0,)),
            slice_sizes=(1,),
            mode=lax.GatherScatterMode.PROMISE_IN_BOUNDS)

    emap = (lanes % 8) * 2
    omap = emap + 1
    half = lanes < 8

    def remap(w, _):
        for h in range(WIN // 32):
            wa = winner_v[pl.ds(w * WIN + h * 32, 16)]
            wb = winner_v[pl.ds(w * WIN + h * 32 + 16, 16)]
            ra = (base + w * WIN + h * 32) + lanes
            rb = ra + 16
            ia = jnp.where(wa < 0, ra, wa)
            ib = jnp.where(wb < 0, rb, wb)
            evens = jnp.where(half, _dg(ia, emap), _dg(ib, emap))
            odds = jnp.where(half, _dg(ia, omap), _dg(ib, omap))
            ibuf[pl.ds(w * WIN + h * 16, 16)] = evens
            ibuf[pl.ds(w * WIN + WIN // 2 + h * 16, 16)] = odds
            vbuf[pl.ds(w * WIN + h * 16, 16)] = jnp.where(
                half, _dg(wa, emap), _dg(wb, emap))
            vbuf[pl.ds(w * WIN + WIN // 2 + h * 16, 16)] = jnp.where(
                half, _dg(wa, omap), _dg(wb, omap))
        return 0
    lax.fori_loop(0, PER_W // WIN, remap, 0)

    scope_g = jax.named_scope("phase_b_gather")
    scope_g.__enter__()
    nwin = nrows // WIN

    def gissue(w, p):
        pltpu.async_copy(pe_hbm.at[ibuf.at[pl.ds(w * WIN, WIN)]], bufs[p],
                         gsems[p])

    def gwait(p):
        pltpu.make_async_copy(pe_hbm.at[ibuf.at[pl.ds(0, WIN)]], bufs[p],
                              gsems[p]).wait()

    def sissue(w, p):
        node_st = pl.multiple_of((base + w * WIN) // 2, 8)
        pltpu.async_copy(bufs[p].at[pl.ds(0, WIN // 2), :],
                         out_hbm.at[pl.ds(node_st, WIN // 2), pl.ds(0, D)],
                         ssems[p])
        pltpu.async_copy(bufs[p].at[pl.ds(WIN // 2, WIN // 2), :],
                         out_hbm.at[pl.ds(node_st, WIN // 2), pl.ds(D, D)],
                         ssems[p])

    def swait(p):
        for _ in range(2):
            pltpu.make_async_copy(
                bufs[p].at[pl.ds(0, WIN // 2), :],
                out_hbm.at[pl.ds(0, WIN // 2), pl.ds(0, D)],
                ssems[p]).wait()

    def zerofix(w, p):
        buf = bufs[p]
        for c in range(WIN // 16):
            wv = vbuf[pl.ds(w * WIN + c * 16, 16)]
            bits = jnp.sum(jnp.where(wv < 0, jnp.int32(1), jnp.int32(0))
                           << lanes)

            @pl.when(bits != 0)
            def _():
                def zrow(l, _):
                    @pl.when(((bits >> l) & 1) == 1)
                    def _():
                        for k in range(D // 16):
                            buf[c * 16 + l, pl.ds(k * 16, 16)] = zeros16
                    return 0
                lax.fori_loop(0, 16, zrow, 0)

    for p in range(NBUF - 1):
        gissue(p, p)

    def pipe(i, _):
        for p in range(NBUF):
            w = i * NBUF + p

            @pl.when(w < nwin)
            def _(w=w, p=p):
                gwait(p)
                q = (p + NBUF - 1) % NBUF

                @pl.when(w + NBUF - 1 < nwin)
                def _():
                    @pl.when(w >= 1)
                    def _():
                        swait(q)
                    gissue(w + NBUF - 1, q)
                zerofix(w, p)
                sissue(w, p)
        return 0
    lax.fori_loop(0, (PER_W // WIN + NBUF - 1) // NBUF, pipe, 0)
    for p in range(NBUF):
        swait(p)
    scope_g.__exit__(None, None, None)


def _sc_call(keys, pe):
    nodes = keys.shape[0] // 2
    call = functools.partial(
        pl.kernel,
        out_type=jax.ShapeDtypeStruct((nodes, 2 * D), jnp.float32),
        mesh=plsc.VectorSubcoreMesh(core_axis_name="c", subcore_axis_name="s"),
        compiler_params=pltpu.CompilerParams(needs_layout_passes=False),
        scratch_types=[
            pltpu.VMEM((PER_W,), jnp.int32),
            [pltpu.VMEM((KWIN,), jnp.int32)] * 2,
            pltpu.VMEM((PER_W,), jnp.int32),
            pltpu.VMEM((PER_W,), jnp.int32),
            [pltpu.VMEM((WIN, D), jnp.float32)] * NBUF,
            [pltpu.SemaphoreType.DMA] * 2,
            [pltpu.SemaphoreType.DMA] * NBUF,
            [pltpu.SemaphoreType.DMA] * NBUF,
        ],
    )(_sc_kernel)
    return call(keys, pe)


def kernel(x, pe):
    keys = x[:, 0] * 2 + x[:, 1]
    return _sc_call(keys, pe)
